# baseline (speedup 1.0000x reference)
import functools

import jax
import jax.numpy as jnp
from jax import lax
from jax.experimental import pallas as pl
from jax.experimental.pallas import tpu as pltpu
from jax.experimental.pallas import tpu_sc as plsc

_NC = 1
_NS = 8
_NW = _NC * _NS
_CHUNK = 128


@functools.cache
def _make_gather(batch):
    b_w = batch // _NW
    n_ch = b_w // _CHUNK
    mesh = plsc.VectorSubcoreMesh(
        core_axis_name="c", subcore_axis_name="s", num_cores=_NC, num_subcores=_NS
    )

    @functools.partial(
        pl.kernel,
        out_type=jax.ShapeDtypeStruct((batch,), jnp.int32),
        mesh=mesh,
        cost_estimate=pl.CostEstimate(
            flops=0, transcendentals=0,
            bytes_accessed=batch * 64 + batch * 3 * 4,
        ),
        scratch_types=[
            pltpu.VMEM((b_w,), jnp.int32),
            pltpu.VMEM((b_w,), jnp.int32),
            pltpu.SemaphoreType.DMA,
            pltpu.SemaphoreType.DMA,
        ],
    )
    def gather_kernel(ids_hbm, table_hbm, out_hbm, idx_v, vals_v, sem, osem):
        wid = lax.axis_index("s") * _NC + lax.axis_index("c")
        base = wid * b_w
        pltpu.sync_copy(ids_hbm.at[pl.ds(base, b_w)], idx_v)
        copies = [
            pltpu.async_copy(
                table_hbm.at[idx_v.at[pl.ds(j * _CHUNK, _CHUNK)]],
                vals_v.at[pl.ds(j * _CHUNK, _CHUNK)],
                sem,
            )
            for j in range(n_ch)
        ]
        ocopies = []
        for j in range(n_ch):
            copies[j].wait()
            ocopies.append(
                pltpu.async_copy(
                    vals_v.at[pl.ds(j * _CHUNK, _CHUNK)],
                    out_hbm.at[pl.ds(base + j * _CHUNK, _CHUNK)],
                    osem,
                )
            )
        for c in ocopies:
            c.wait()

    return gather_kernel


def kernel(input_ids, features, lookup_table):
    batch = input_ids.shape[0]
    ids = input_ids.astype(jnp.int32)
    out = _make_gather(batch)(ids, lookup_table)
    return (out, features)


# speedup vs baseline: 1.0666x; 1.0666x over previous
import functools

import jax
import jax.numpy as jnp
from jax import lax
from jax.experimental import pallas as pl
from jax.experimental.pallas import tpu as pltpu
from jax.experimental.pallas import tpu_sc as plsc

_NC = 1
_NS = 16
_NW = _NC * _NS
_CHUNK = 128


@functools.cache
def _make_gather(batch):
    b_w = batch // _NW
    n_ch = b_w // _CHUNK
    mesh = plsc.VectorSubcoreMesh(
        core_axis_name="c", subcore_axis_name="s", num_cores=_NC, num_subcores=_NS
    )

    @functools.partial(
        pl.kernel,
        out_type=jax.ShapeDtypeStruct((batch,), jnp.int32),
        mesh=mesh,
        cost_estimate=pl.CostEstimate(
            flops=0, transcendentals=0,
            bytes_accessed=batch * 64 + batch * 3 * 4,
        ),
        scratch_types=[
            pltpu.VMEM((b_w,), jnp.int32),
            pltpu.VMEM((b_w,), jnp.int32),
            pltpu.SemaphoreType.DMA,
            pltpu.SemaphoreType.DMA,
        ],
    )
    def gather_kernel(ids_hbm, table_hbm, out_hbm, idx_v, vals_v, sem, osem):
        wid = lax.axis_index("s") * _NC + lax.axis_index("c")
        base = wid * b_w
        half = b_w // 2
        pltpu.sync_copy(ids_hbm.at[pl.ds(base, half)], idx_v.at[pl.ds(0, half)])
        copies = [
            pltpu.async_copy(
                table_hbm.at[idx_v.at[pl.ds(j * _CHUNK, _CHUNK)]],
                vals_v.at[pl.ds(j * _CHUNK, _CHUNK)],
                sem,
            )
            for j in range(n_ch // 2)
        ]
        pltpu.sync_copy(
            ids_hbm.at[pl.ds(base + half, b_w - half)],
            idx_v.at[pl.ds(half, b_w - half)],
        )
        copies += [
            pltpu.async_copy(
                table_hbm.at[idx_v.at[pl.ds(j * _CHUNK, _CHUNK)]],
                vals_v.at[pl.ds(j * _CHUNK, _CHUNK)],
                sem,
            )
            for j in range(n_ch // 2, n_ch)
        ]
        ocopies = []
        for j in range(n_ch):
            copies[j].wait()
            ocopies.append(
                pltpu.async_copy(
                    vals_v.at[pl.ds(j * _CHUNK, _CHUNK)],
                    out_hbm.at[pl.ds(base + j * _CHUNK, _CHUNK)],
                    osem,
                )
            )
        for c in ocopies:
            c.wait()

    return gather_kernel


def kernel(input_ids, features, lookup_table):
    batch = input_ids.shape[0]
    ids = input_ids.astype(jnp.int32)
    out = _make_gather(batch)(ids, lookup_table)
    return (out, features)
